# baseline (device time: 54942 ns/iter reference)
import jax
import jax.numpy as jnp
from jax import lax
from jax.experimental import pallas as pl
from jax.experimental.pallas import tpu as pltpu

B, S, H, Dh, Dr = 2, 256, 16, 64, 32
D = 1024
DC_SH = 64
SQ = S // 4
F32 = jnp.float32


def kernel(x, Wdkv, Wuk, Wuv, Wq, Wqr, Wkr, Wo):
    def body(
        x_ref, wdkv_ref, wuk_ref, wuv_ref, wq_ref, wqr_ref, wkr_ref, wo_ref,
        out_ref,
        c_ref, c_peer_ref, wuk_peer_ref, wuv_peer_ref, o_ref,
        send_sems, recv_sems,
    ):
        my_x = lax.axis_index("x")
        my_y = lax.axis_index("y")
        my_z = lax.axis_index("z")
        x_peer = (1 - my_x, my_y, my_z)
        y_peer = (my_x, 1 - my_y, my_z)
        z_peer = (my_x, my_y, 1 - my_z)
        qoff = SQ * (2 * my_y + my_x)
        yp_off = SQ * (2 * (1 - my_y) + my_x)

        barrier_sem = pltpu.get_barrier_semaphore()
        for peer in (x_peer, y_peer, z_peer):
            pl.semaphore_signal(
                barrier_sem, inc=1, device_id=peer,
                device_id_type=pl.DeviceIdType.MESH,
            )
        pl.semaphore_wait(barrier_sem, 3)

        rdma_wuk = pltpu.make_async_remote_copy(
            src_ref=wuk_ref, dst_ref=wuk_peer_ref,
            send_sem=send_sems.at[0], recv_sem=recv_sems.at[0],
            device_id=z_peer, device_id_type=pl.DeviceIdType.MESH,
        )
        rdma_wuk.start()
        rdma_wuv = pltpu.make_async_remote_copy(
            src_ref=wuv_ref, dst_ref=wuv_peer_ref,
            send_sem=send_sems.at[1], recv_sem=recv_sems.at[1],
            device_id=z_peer, device_id_type=pl.DeviceIdType.MESH,
        )
        rdma_wuv.start()

        x2 = jnp.reshape(x_ref[...], (B * S, D))
        c_ref[...] = jnp.dot(x2, wdkv_ref[...], preferred_element_type=F32)
        rdma_c = pltpu.make_async_remote_copy(
            src_ref=c_ref, dst_ref=c_peer_ref,
            send_sem=send_sems.at[2], recv_sem=recv_sems.at[2],
            device_id=z_peer, device_id_type=pl.DeviceIdType.MESH,
        )
        rdma_c.start()

        scale = (Dh + Dr) ** -0.5
        xq = jnp.reshape(x_ref[:, pl.ds(qoff, SQ), :], (B * SQ, D))
        q2 = jnp.dot(xq, wq_ref[...], preferred_element_type=F32) * scale
        qr2 = jnp.dot(xq, wqr_ref[...], preferred_element_type=F32) * scale
        kr2 = jnp.dot(x2, wkr_ref[...], preferred_element_type=F32)

        rdma_wuk.wait()
        rdma_wuv.wait()
        rdma_c.wait()

        cf = jnp.concatenate([c_ref[...], c_peer_ref[...]], axis=1)
        wukf = jnp.concatenate([wuk_ref[...], wuk_peer_ref[...]], axis=0)
        wuvf = jnp.concatenate([wuv_ref[...], wuv_peer_ref[...]], axis=0)
        k2 = jnp.dot(cf, wukf, preferred_element_type=F32)
        v2 = jnp.dot(cf, wuvf, preferred_element_type=F32)

        dn = (((1,), (1,)), ((), ()))
        for b in range(B):
            kb = k2[b * S:(b + 1) * S]
            vb = v2[b * S:(b + 1) * S]
            krb = kr2[b * S:(b + 1) * S]
            qb = q2[b * SQ:(b + 1) * SQ]
            qrb = qr2[b * SQ:(b + 1) * SQ]
            for h in range(H):
                q = qb[:, h * Dh:(h + 1) * Dh]
                k = kb[:, h * Dh:(h + 1) * Dh]
                qr = qrb[:, h * Dr:(h + 1) * Dr]
                s = (
                    lax.dot_general(q, k, dn, preferred_element_type=F32)
                    + lax.dot_general(qr, krb, dn, preferred_element_type=F32)
                )
                m = jnp.max(s, axis=-1, keepdims=True)
                p = jnp.exp(s - m)
                p = p / jnp.sum(p, axis=-1, keepdims=True)
                o_ref[b, :, h * Dh:(h + 1) * Dh] = jnp.dot(
                    p, vb[:, h * Dh:(h + 1) * Dh], preferred_element_type=F32
                )
            out_ref[b, pl.ds(qoff, SQ), :] = jnp.dot(
                o_ref[b], wo_ref[...], preferred_element_type=F32
            )

        rA_x = pltpu.make_async_remote_copy(
            src_ref=out_ref.at[:, pl.ds(qoff, SQ), :],
            dst_ref=out_ref.at[:, pl.ds(qoff, SQ), :],
            send_sem=send_sems.at[3], recv_sem=recv_sems.at[3],
            device_id=x_peer, device_id_type=pl.DeviceIdType.MESH,
        )
        rA_x.start()
        rA_y = pltpu.make_async_remote_copy(
            src_ref=out_ref.at[:, pl.ds(qoff, SQ), :],
            dst_ref=out_ref.at[:, pl.ds(qoff, SQ), :],
            send_sem=send_sems.at[4], recv_sem=recv_sems.at[4],
            device_id=y_peer, device_id_type=pl.DeviceIdType.MESH,
        )
        rA_y.start()
        rA_y.wait()

        rB_x = pltpu.make_async_remote_copy(
            src_ref=out_ref.at[:, pl.ds(yp_off, SQ), :],
            dst_ref=out_ref.at[:, pl.ds(yp_off, SQ), :],
            send_sem=send_sems.at[5], recv_sem=recv_sems.at[5],
            device_id=x_peer, device_id_type=pl.DeviceIdType.MESH,
        )
        rB_x.start()
        rA_x.wait()
        rB_x.wait()

    vmem = pl.BlockSpec(memory_space=pltpu.VMEM)
    return pl.pallas_call(
        body,
        out_shape=jax.ShapeDtypeStruct((B, S, D), F32),
        in_specs=[vmem] * 8,
        out_specs=vmem,
        scratch_shapes=[
            pltpu.VMEM((B * S, DC_SH), F32),
            pltpu.VMEM((B * S, DC_SH), F32),
            pltpu.VMEM((DC_SH, D), F32),
            pltpu.VMEM((DC_SH, D), F32),
            pltpu.VMEM((B, SQ, H * Dh), F32),
            pltpu.SemaphoreType.DMA((6,)),
            pltpu.SemaphoreType.DMA((6,)),
        ],
        compiler_params=pltpu.CompilerParams(collective_id=0),
    )(x, Wdkv, Wuk, Wuv, Wq, Wqr, Wkr, Wo)


# device time: 31108 ns/iter; 1.7662x vs baseline; 1.7662x over previous
import jax
import jax.numpy as jnp
from jax import lax
from jax.experimental import pallas as pl
from jax.experimental.pallas import tpu as pltpu

B, S, H, Dh, Dr = 2, 256, 16, 64, 32
D = 1024
DC_SH = 64
F32 = jnp.float32
BF16 = jnp.bfloat16


def kernel(x, Wdkv, Wuk, Wuv, Wq, Wqr, Wkr, Wo):
    def body(
        x_ref, wdkv_ref, wuk_ref, wuv_ref, wq_ref, wqr_ref, wkr_ref, wo_ref,
        out_ref,
        c_ref, c_peer_ref, wukbf_ref, wuvbf_ref, wuk_peer_ref, wuv_peer_ref,
        o_ref, send_sems, recv_sems,
    ):
        my_x = lax.axis_index("x")
        my_y = lax.axis_index("y")
        my_z = lax.axis_index("z")
        z_peer = (my_x, my_y, 1 - my_z)

        barrier_sem = pltpu.get_barrier_semaphore()
        pl.semaphore_signal(
            barrier_sem, inc=1, device_id=z_peer,
            device_id_type=pl.DeviceIdType.MESH,
        )
        pl.semaphore_wait(barrier_sem, 1)

        wukbf_ref[...] = wuk_ref[...].astype(BF16)
        wuvbf_ref[...] = wuv_ref[...].astype(BF16)
        rdma_wuk = pltpu.make_async_remote_copy(
            src_ref=wukbf_ref, dst_ref=wuk_peer_ref,
            send_sem=send_sems.at[0], recv_sem=recv_sems.at[0],
            device_id=z_peer, device_id_type=pl.DeviceIdType.MESH,
        )
        rdma_wuk.start()
        rdma_wuv = pltpu.make_async_remote_copy(
            src_ref=wuvbf_ref, dst_ref=wuv_peer_ref,
            send_sem=send_sems.at[1], recv_sem=recv_sems.at[1],
            device_id=z_peer, device_id_type=pl.DeviceIdType.MESH,
        )
        rdma_wuv.start()

        xbf = jnp.reshape(x_ref[...], (B * S, D)).astype(BF16)
        c_ref[...] = jnp.dot(
            xbf, wdkv_ref[...].astype(BF16), preferred_element_type=F32
        ).astype(BF16)
        rdma_c = pltpu.make_async_remote_copy(
            src_ref=c_ref, dst_ref=c_peer_ref,
            send_sem=send_sems.at[2], recv_sem=recv_sems.at[2],
            device_id=z_peer, device_id_type=pl.DeviceIdType.MESH,
        )
        rdma_c.start()

        scale = (Dh + Dr) ** -0.5 * 1.4426950408889634
        q2 = (
            jnp.dot(xbf, wq_ref[...].astype(BF16), preferred_element_type=F32)
            * scale
        ).astype(BF16)
        qr2 = (
            jnp.dot(xbf, wqr_ref[...].astype(BF16), preferred_element_type=F32)
            * scale
        ).astype(BF16)
        kr2 = jnp.dot(
            xbf, wkr_ref[...].astype(BF16), preferred_element_type=F32
        ).astype(BF16)

        wobf = wo_ref[...].astype(BF16)
        k2a = jnp.dot(c_ref[...], wukbf_ref[...], preferred_element_type=F32)
        v2a = jnp.dot(c_ref[...], wuvbf_ref[...], preferred_element_type=F32)

        rdma_wuk.wait()
        rdma_wuv.wait()
        rdma_c.wait()

        k2 = (
            k2a
            + jnp.dot(c_peer_ref[...], wuk_peer_ref[...], preferred_element_type=F32)
        ).astype(BF16)
        v2 = (
            v2a
            + jnp.dot(c_peer_ref[...], wuv_peer_ref[...], preferred_element_type=F32)
        ).astype(BF16)

        dn = (((1,), (1,)), ((), ()))
        for b in range(B):
            kb = k2[b * S:(b + 1) * S]
            vb = v2[b * S:(b + 1) * S]
            krb = kr2[b * S:(b + 1) * S]
            qb = q2[b * S:(b + 1) * S]
            qrb = qr2[b * S:(b + 1) * S]
            qr_stack = jnp.concatenate(
                [qrb[:, h * Dr:(h + 1) * Dr] for h in range(H)], axis=0
            )
            sr_all = lax.dot_general(
                qr_stack, krb, dn, preferred_element_type=F32
            )
            for h in range(H):
                q = qb[:, h * Dh:(h + 1) * Dh]
                k = kb[:, h * Dh:(h + 1) * Dh]
                s = lax.dot_general(
                    q, k, dn, preferred_element_type=F32
                ) + sr_all[h * S:(h + 1) * S]
                p = jnp.exp2(s)
                r = 1.0 / jnp.sum(p, axis=-1, keepdims=True)
                ou = jnp.dot(
                    p.astype(BF16), vb[:, h * Dh:(h + 1) * Dh],
                    preferred_element_type=F32,
                )
                o_ref[b, :, h * Dh:(h + 1) * Dh] = (ou * r).astype(BF16)
        out_ref[...] = jnp.reshape(
            jnp.dot(
                jnp.reshape(o_ref[...], (B * S, H * Dh)),
                wobf,
                preferred_element_type=F32,
            ),
            (B, S, D),
        )

    vmem = pl.BlockSpec(memory_space=pltpu.VMEM)
    return pl.pallas_call(
        body,
        out_shape=jax.ShapeDtypeStruct((B, S, D), F32),
        in_specs=[vmem] * 8,
        out_specs=vmem,
        scratch_shapes=[
            pltpu.VMEM((B * S, DC_SH), BF16),
            pltpu.VMEM((B * S, DC_SH), BF16),
            pltpu.VMEM((DC_SH, D), BF16),
            pltpu.VMEM((DC_SH, D), BF16),
            pltpu.VMEM((DC_SH, D), BF16),
            pltpu.VMEM((DC_SH, D), BF16),
            pltpu.VMEM((B, S, H * Dh), BF16),
            pltpu.SemaphoreType.DMA((3,)),
            pltpu.SemaphoreType.DMA((3,)),
        ],
        compiler_params=pltpu.CompilerParams(collective_id=0),
    )(x, Wdkv, Wuk, Wuv, Wq, Wqr, Wkr, Wo)
